# Initial kernel scaffold; baseline (speedup 1.0000x reference)
import jax
import jax.numpy as jnp
from jax.experimental import pallas as pl


def kernel(points, params):
    raise NotImplementedError("write your pallas kernel here")


# speedup vs baseline: 3.8277x; 3.8277x over previous
import functools

import jax
import jax.numpy as jnp
import numpy as np
from jax.experimental import pallas as pl
from jax.experimental.pallas import tpu as pltpu

_BN = np.float32(1.0 / np.sqrt(1.0 + 1e-5))
_SA_CFG = [(1024, 0.1, 32), (256, 0.2, 32), (64, 0.4, 32), (16, 0.8, 32)]


def _iota_f32(shape, dim):
    return jax.lax.broadcasted_iota(jnp.int32, shape, dim).astype(jnp.float32)


def _first_min_idx(vals, iota_lanes, big):
    v = jnp.min(vals, axis=1, keepdims=True)
    idx = jnp.min(jnp.where(vals <= v, iota_lanes, big), axis=1, keepdims=True)
    return v, idx


def _fps_body(xyz_t, xyz_rows, n, m):
    x = xyz_t[...]
    xr = xyz_rows[...]
    lane_iota = jax.lax.broadcasted_iota(jnp.int32, (1, n), 1)
    row_iota = jax.lax.broadcasted_iota(jnp.int32, (n, 1), 0)
    lane_m = jax.lax.broadcasted_iota(jnp.int32, (1, m), 1)
    row_m = jax.lax.broadcasted_iota(jnp.int32, (m, 1), 0)

    def step(t, carry):
        dists, far, cent_t, cent_rows = carry
        onehot = (lane_iota == far).astype(jnp.float32)
        c = jnp.sum(x * onehot, axis=1, keepdims=True)
        cent_t = jnp.where(lane_m == t, c, cent_t)
        rowhot = (row_iota == far).astype(jnp.float32)
        c_row = jnp.sum(xr * rowhot, axis=0, keepdims=True)
        cent_rows = jnp.where(row_m == t, c_row, cent_rows)
        d = jnp.sum((x - c) ** 2, axis=0, keepdims=True)
        dists = jnp.minimum(dists, d)
        mx = jnp.max(dists)
        far2 = jnp.min(jnp.where(dists >= mx, lane_iota, n))
        return dists, far2, cent_t, cent_rows

    init = (jnp.full((1, n), 1e10, jnp.float32), jnp.int32(0),
            jnp.zeros((3, m), jnp.float32), jnp.zeros((m, 3), jnp.float32))
    _, _, cent_t, cent_rows = jax.lax.fori_loop(0, m, step, init)
    return cent_t, cent_rows


def _ball_query_body(xyz_t, cent_rows, n, m, radius, k, mc, gidx_ref):
    x = xyz_t[...]
    xx = jnp.sum(x * x, axis=0, keepdims=True)
    lane_iota = _iota_f32((1, n), 1)
    lane_k = jax.lax.broadcasted_iota(jnp.int32, (1, k), 1)
    r2 = np.float32(radius * radius)
    nf = np.float32(n)
    for c0 in range(0, m, mc):
        cent = jax.lax.slice(cent_rows, (c0, 0), (c0 + mc, 3))
        cc = jnp.sum(cent * cent, axis=1, keepdims=True)
        ab = jax.lax.dot_general(
            cent, x, (((1,), (0,)), ((), ())),
            preferred_element_type=jnp.float32)
        d2 = jnp.maximum(cc + xx - 2.0 * ab, 0.0)
        cand = jnp.where(d2 <= r2, lane_iota, nf)
        v0 = jnp.min(cand, axis=1, keepdims=True)

        def kstep(kk, carry):
            cand, slots = carry
            v = jnp.min(cand, axis=1, keepdims=True)
            slot = jnp.where(v < nf, v, v0)
            slots = jnp.where(lane_k == kk, slot, slots)
            return jnp.where(cand == v, np.float32(1e9), cand), slots

        _, slots = jax.lax.fori_loop(
            0, k, kstep, (cand, jnp.zeros((mc, k), jnp.float32)))
        gidx_ref[pl.ds(c0, mc), :] = slots


def _sa_kernel(n, m, radius, k, g_tile, mc, factorized,
               xyz_t, xyz_rows, xsrc, w1, w2, w3,
               cent_t_ref, cent_rows_ref, feat_ref, gidx_ref):
    cent_t, cent_rows = _fps_body(xyz_t, xyz_rows, n, m)
    cent_t_ref[...] = cent_t
    cent_rows_ref[...] = cent_rows
    _ball_query_body(xyz_t, cent_rows, n, m, radius, k, mc, gidx_ref)

    w1v, w2v, w3v = w1[...], w2[...], w3[...]
    w1_3 = w1v[:3, :]
    xs = xsrc[...]

    def tile(t, _):
        t0 = pl.multiple_of(t * g_tile, g_tile)
        idx = gidx_ref[pl.ds(t0, g_tile), :]
        rows = g_tile * k
        if factorized:
            hi = jnp.floor(idx * np.float32(1.0 / 128.0))
            lo = idx - hi * 128.0
            oh_lo = (lo[:, :, None] == _iota_f32((1, 1, 128), 2))
            oh_lo = oh_lo.astype(jnp.float32).reshape(rows, 128)
            y = jnp.dot(oh_lo, xs, preferred_element_type=jnp.float32)
            blk = jnp.floor(_iota_f32((1, 1, 256), 2) * np.float32(1.0 / 8.0))
            mask = (blk == hi[:, :, None]).astype(jnp.float32)
            z = y * mask.reshape(rows, 256)
            z = z[:, :128] + z[:, 128:]
            z = z[:, :64] + z[:, 64:]
            z = z[:, :32] + z[:, 32:]
            z = z[:, :16] + z[:, 16:]
            g = z[:, :8] + z[:, 8:]
        else:
            oh = (idx[:, :, None] == _iota_f32((1, 1, n), 2))
            oh = oh.astype(jnp.float32).reshape(rows, n)
            g = jnp.dot(oh, xs, preferred_element_type=jnp.float32)

        cent = cent_rows_ref[pl.ds(t0, g_tile), :]
        cw = jnp.dot(cent, w1_3, preferred_element_type=jnp.float32)
        h = jnp.dot(g, w1v, preferred_element_type=jnp.float32)
        h = h.reshape(g_tile, k, -1) - cw[:, None, :]
        h = jax.nn.relu(h * _BN).reshape(rows, -1)
        h = jax.nn.relu(jnp.dot(h, w2v, preferred_element_type=jnp.float32) * _BN)
        h = jax.nn.relu(jnp.dot(h, w3v, preferred_element_type=jnp.float32) * _BN)
        h = jnp.max(h.reshape(g_tile, k, -1), axis=1)
        feat_ref[pl.ds(t0, g_tile), :] = h
        return 0

    jax.lax.fori_loop(0, m // g_tile, tile, 0)


def _fp_kernel(s, d, nw, head, sxyz_rows, dxyz_t, dfeat, sfeat, *refs):
    ws_refs, out_ref = refs[:-1], refs[-1]
    sx = sxyz_rows[...]
    dx = dxyz_t[...]
    cc_s = jnp.sum(sx * sx, axis=1, keepdims=True)
    cc_d = jnp.sum(dx * dx, axis=0, keepdims=True)
    ab = jax.lax.dot_general(sx, dx, (((1,), (0,)), ((), ())),
                             preferred_element_type=jnp.float32)
    d2 = jnp.maximum(cc_s + cc_d - 2.0 * ab, 0.0)
    lane_iota = _iota_f32((1, d), 1)
    big = np.float32(d)
    vs, idxs = [], []
    for _ in range(3):
        v, idx = _first_min_idx(d2, lane_iota, big)
        vs.append(v)
        idxs.append(idx)
        d2 = jnp.where(lane_iota == idx, np.float32(1e9), d2)
    ws = [1.0 / (v + np.float32(1e-8)) for v in vs]
    wsum = ws[0] + ws[1] + ws[2]
    df = dfeat[...]
    interp = None
    for v_idx, w in zip(idxs, ws):
        oh = (lane_iota == v_idx).astype(jnp.float32)
        gathered = jnp.dot(oh, df, preferred_element_type=jnp.float32)
        term = gathered * (w / wsum)
        interp = term if interp is None else interp + term
    f = jnp.concatenate([interp, sfeat[...]], axis=1)
    for i in range(nw):
        wv = ws_refs[i][...]
        f = jax.nn.relu(jnp.dot(f, wv, preferred_element_type=jnp.float32) * _BN)
    if head:
        seg_w, logit_w, logit_b = ws_refs[nw][...], ws_refs[nw + 1][...], ws_refs[nw + 2][...]
        f = jax.nn.relu(jnp.dot(f, seg_w, preferred_element_type=jnp.float32) * _BN)
        f = jnp.dot(f, logit_w, preferred_element_type=jnp.float32) + logit_b
    out_ref[...] = f


def _run_sa(level, xyz_t, xyz_rows, xsrc, w1, w2, w3):
    m, radius, k, n = *_SA_CFG[level], xyz_t.shape[-1]
    factorized = level == 0
    g_tile = 64 if factorized else 16
    mc = min(m, 256)
    c3 = w3.shape[-1]
    fn = functools.partial(_sa_kernel, n, m, radius, k, g_tile, mc, factorized)
    call = pl.pallas_call(
        fn,
        out_shape=(
            jax.ShapeDtypeStruct((3, m), jnp.float32),
            jax.ShapeDtypeStruct((m, 3), jnp.float32),
            jax.ShapeDtypeStruct((m, c3), jnp.float32),
        ),
        scratch_shapes=[pltpu.VMEM((m, k), jnp.float32)],
    )
    return jax.vmap(call, in_axes=(0, 0, 0, None, None, None))(
        xyz_t, xyz_rows, xsrc, w1, w2, w3)


def _run_fp(sxyz_rows, dxyz_t, dfeat, sfeat, ws, head_ws=None):
    s, d = sxyz_rows.shape[-2], dxyz_t.shape[-1]
    nw = len(ws)
    head = head_ws is not None
    all_ws = list(ws) + (list(head_ws) if head else [])
    cout = 13 if head else ws[-1].shape[-1]
    fn = functools.partial(_fp_kernel, s, d, nw, head)
    call = pl.pallas_call(
        fn,
        out_shape=jax.ShapeDtypeStruct((s, cout), jnp.float32),
    )
    in_axes = (0, 0, 0, 0) + (None,) * len(all_ws)
    return jax.vmap(call, in_axes=in_axes)(
        sxyz_rows, dxyz_t, dfeat, sfeat, *all_ws)


def kernel(points, params):
    xyz_rows = jnp.transpose(points[:, 0:3, :], (0, 2, 1))
    xyz_t = points[:, 0:3, :]
    feat_rows = jnp.transpose(points[:, 3:, :], (0, 2, 1))

    inter_xyz_rows = [xyz_rows]
    inter_xyz_t = [xyz_t]
    inter_feat = [jnp.transpose(points, (0, 2, 1))]

    cur_xyz_t, cur_xyz_rows, cur_feat = xyz_t, xyz_rows, feat_rows
    for level in range(4):
        xsrc = jnp.concatenate([cur_xyz_rows, cur_feat], axis=-1)
        if level == 0:
            b, n, _ = xsrc.shape
            xsrc = jnp.pad(xsrc, ((0, 0), (0, 0), (0, 2)))
            xsrc = xsrc.reshape(b, 32, 128, 8).transpose(0, 2, 1, 3)
            xsrc = xsrc.reshape(b, 128, 256)
            w1, w2, w3 = params['sa'][0]
            w1 = jnp.pad(w1, ((0, 2), (0, 0)))
        else:
            w1, w2, w3 = params['sa'][level]
        cent_t, cent_rows, feat = _run_sa(
            level, cur_xyz_t, cur_xyz_rows, xsrc, w1, w2, w3)
        cur_xyz_t, cur_xyz_rows, cur_feat = cent_t, cent_rows, feat
        inter_xyz_rows.append(cent_rows)
        inter_xyz_t.append(cent_t)
        inter_feat.append(feat)

    dfeat = jnp.concatenate([cur_xyz_rows, cur_feat], axis=-1)
    dxyz_t = cur_xyz_t
    for i in range(4):
        sxyz_rows = inter_xyz_rows[-1 - i]
        sfeat = inter_feat[-1 - i]
        head_ws = None
        if i == 3:
            head_ws = [params['seg'][0], params['logit_w'],
                       params['logit_b'].reshape(1, 13)]
        dfeat = _run_fp(sxyz_rows, dxyz_t, dfeat, sfeat, params['fp'][i],
                        head_ws)
        dxyz_t = inter_xyz_t[-1 - i]
    return jnp.transpose(dfeat, (0, 2, 1))


# speedup vs baseline: 5.9181x; 1.5461x over previous
import functools

import jax
import jax.numpy as jnp
import numpy as np
from jax.experimental import pallas as pl
from jax.experimental.pallas import tpu as pltpu

_BN = np.float32(1.0 / np.sqrt(1.0 + 1e-5))
_SA_CFG = [(1024, 0.1, 32), (256, 0.2, 32), (64, 0.4, 32), (16, 0.8, 32)]


def _iota_f32(shape, dim):
    return jax.lax.broadcasted_iota(jnp.int32, shape, dim).astype(jnp.float32)


def _first_min_idx(vals, iota_lanes, big):
    v = jnp.min(vals, axis=1, keepdims=True)
    idx = jnp.min(jnp.where(vals <= v, iota_lanes, big), axis=1, keepdims=True)
    return v, idx


def _fps_kernel(n, m, xyz_f, cent_ref):
    x = xyz_f[...]
    x0, x1, x2 = x[0], x[1], x[2]
    r, l = x0.shape
    cr, cl = cent_ref.shape[1], cent_ref.shape[2]
    flat = (jax.lax.broadcasted_iota(jnp.int32, (r, l), 0) * l
            + jax.lax.broadcasted_iota(jnp.int32, (r, l), 1))
    cflat = (jax.lax.broadcasted_iota(jnp.int32, (cr, cl), 0) * cl
             + jax.lax.broadcasted_iota(jnp.int32, (cr, cl), 1))

    def step(t, carry):
        dists, far, cx, cy, cz = carry
        oh = (flat == far).astype(jnp.float32)
        c0 = jnp.sum(x0 * oh)
        c1 = jnp.sum(x1 * oh)
        c2 = jnp.sum(x2 * oh)
        cx = jnp.where(cflat == t, c0, cx)
        cy = jnp.where(cflat == t, c1, cy)
        cz = jnp.where(cflat == t, c2, cz)
        d = (x0 - c0) ** 2 + (x1 - c1) ** 2 + (x2 - c2) ** 2
        dists = jnp.minimum(dists, d)
        mx = jnp.max(dists)
        far2 = jnp.min(jnp.where(dists >= mx, flat, n))
        return dists, far2, cx, cy, cz

    zc = jnp.zeros((cr, cl), jnp.float32)
    init = (jnp.full((r, l), 1e10, jnp.float32), jnp.int32(0), zc, zc, zc)
    _, _, cx, cy, cz = jax.lax.fori_loop(0, m, step, init)
    cent_ref[...] = jnp.stack([cx, cy, cz], axis=0)


def _run_fps(level, xyz_t):
    b, _, n = xyz_t.shape
    m = _SA_CFG[level][0]
    l = min(n, 128)
    cl = min(m, 128)
    xyz_f = xyz_t.reshape(b, 3, n // l, l)
    call = pl.pallas_call(
        functools.partial(_fps_kernel, n, m),
        out_shape=jax.ShapeDtypeStruct((3, m // cl, cl), jnp.float32),
    )
    cent = jax.vmap(call)(xyz_f)
    cent_t = cent.reshape(b, 3, m)
    return cent_t, jnp.transpose(cent_t, (0, 2, 1))


def _ball_query_body(xyz_t, cent_rows, n, m, radius, k, mc, gidx_ref):
    x = xyz_t[...]
    xx = jnp.sum(x * x, axis=0, keepdims=True)
    lane_iota = _iota_f32((1, n), 1)
    lane_k = jax.lax.broadcasted_iota(jnp.int32, (1, k), 1)
    r2 = np.float32(radius * radius)
    nf = np.float32(n)
    for c0 in range(0, m, mc):
        cent = jax.lax.slice(cent_rows, (c0, 0), (c0 + mc, 3))
        cc = jnp.sum(cent * cent, axis=1, keepdims=True)
        ab = jax.lax.dot_general(
            cent, x, (((1,), (0,)), ((), ())),
            preferred_element_type=jnp.float32)
        d2 = jnp.maximum(cc + xx - 2.0 * ab, 0.0)
        cand = jnp.where(d2 <= r2, lane_iota, nf)
        v0 = jnp.min(cand, axis=1, keepdims=True)

        def kstep(kk, carry):
            cand, slots = carry
            v = jnp.min(cand, axis=1, keepdims=True)
            slot = jnp.where(v < nf, v, v0)
            slots = jnp.where(lane_k == kk, slot, slots)
            return jnp.where(cand == v, np.float32(1e9), cand), slots

        _, slots = jax.lax.fori_loop(
            0, k, kstep, (cand, jnp.zeros((mc, k), jnp.float32)))
        gidx_ref[pl.ds(c0, mc), :] = slots


def _sa_kernel(n, m, radius, k, g_tile, mc, factorized,
               xyz_t, xsrc, cent_rows_in, w1, w2, w3,
               feat_ref, gidx_ref):
    cent_rows = cent_rows_in[...]
    _ball_query_body(xyz_t, cent_rows, n, m, radius, k, mc, gidx_ref)

    w1v, w2v, w3v = w1[...], w2[...], w3[...]
    w1_3 = w1v[:3, :]
    xs = xsrc[...]

    def tile(t, _):
        t0 = pl.multiple_of(t * g_tile, g_tile)
        idx = gidx_ref[pl.ds(t0, g_tile), :]
        rows = g_tile * k
        if factorized:
            hi = jnp.floor(idx * np.float32(1.0 / 128.0))
            lo = idx - hi * 128.0
            oh_lo = (lo[:, :, None] == _iota_f32((1, 1, 128), 2))
            oh_lo = oh_lo.astype(jnp.float32).reshape(rows, 128)
            y = jnp.dot(oh_lo, xs, preferred_element_type=jnp.float32)
            blk = jnp.floor(_iota_f32((1, 1, 256), 2) * np.float32(1.0 / 8.0))
            mask = (blk == hi[:, :, None]).astype(jnp.float32)
            z = y * mask.reshape(rows, 256)
            z = z[:, :128] + z[:, 128:]
            z = z[:, :64] + z[:, 64:]
            z = z[:, :32] + z[:, 32:]
            z = z[:, :16] + z[:, 16:]
            g = z[:, :8] + z[:, 8:]
        else:
            oh = (idx[:, :, None] == _iota_f32((1, 1, n), 2))
            oh = oh.astype(jnp.float32).reshape(rows, n)
            g = jnp.dot(oh, xs, preferred_element_type=jnp.float32)

        cent = cent_rows_in[pl.ds(t0, g_tile), :]
        cw = jnp.dot(cent, w1_3, preferred_element_type=jnp.float32)
        h = jnp.dot(g, w1v, preferred_element_type=jnp.float32)
        h = h.reshape(g_tile, k, -1) - cw[:, None, :]
        h = jax.nn.relu(h * _BN).reshape(rows, -1)
        h = jax.nn.relu(jnp.dot(h, w2v, preferred_element_type=jnp.float32) * _BN)
        h = jax.nn.relu(jnp.dot(h, w3v, preferred_element_type=jnp.float32) * _BN)
        h = jnp.max(h.reshape(g_tile, k, -1), axis=1)
        feat_ref[pl.ds(t0, g_tile), :] = h
        return 0

    jax.lax.fori_loop(0, m // g_tile, tile, 0)


def _fp_kernel(s, d, nw, head, sxyz_rows, dxyz_t, dfeat, sfeat, *refs):
    ws_refs, out_ref = refs[:-1], refs[-1]
    sx = sxyz_rows[...]
    dx = dxyz_t[...]
    cc_s = jnp.sum(sx * sx, axis=1, keepdims=True)
    cc_d = jnp.sum(dx * dx, axis=0, keepdims=True)
    ab = jax.lax.dot_general(sx, dx, (((1,), (0,)), ((), ())),
                             preferred_element_type=jnp.float32)
    d2 = jnp.maximum(cc_s + cc_d - 2.0 * ab, 0.0)
    lane_iota = _iota_f32((1, d), 1)
    big = np.float32(d)
    vs, idxs = [], []
    for _ in range(3):
        v, idx = _first_min_idx(d2, lane_iota, big)
        vs.append(v)
        idxs.append(idx)
        d2 = jnp.where(lane_iota == idx, np.float32(1e9), d2)
    ws = [1.0 / (v + np.float32(1e-8)) for v in vs]
    wsum = ws[0] + ws[1] + ws[2]
    df = dfeat[...]
    interp = None
    for v_idx, w in zip(idxs, ws):
        oh = (lane_iota == v_idx).astype(jnp.float32)
        gathered = jnp.dot(oh, df, preferred_element_type=jnp.float32)
        term = gathered * (w / wsum)
        interp = term if interp is None else interp + term
    f = jnp.concatenate([interp, sfeat[...]], axis=1)
    for i in range(nw):
        wv = ws_refs[i][...]
        f = jax.nn.relu(jnp.dot(f, wv, preferred_element_type=jnp.float32) * _BN)
    if head:
        seg_w, logit_w, logit_b = ws_refs[nw][...], ws_refs[nw + 1][...], ws_refs[nw + 2][...]
        f = jax.nn.relu(jnp.dot(f, seg_w, preferred_element_type=jnp.float32) * _BN)
        f = jnp.dot(f, logit_w, preferred_element_type=jnp.float32) + logit_b
    out_ref[...] = f


def _run_sa(level, xyz_t, xsrc, cent_rows, w1, w2, w3):
    m, radius, k, n = *_SA_CFG[level], xyz_t.shape[-1]
    factorized = level == 0
    g_tile = 64 if factorized else 16
    mc = min(m, 256)
    c3 = w3.shape[-1]
    fn = functools.partial(_sa_kernel, n, m, radius, k, g_tile, mc, factorized)
    call = pl.pallas_call(
        fn,
        out_shape=jax.ShapeDtypeStruct((m, c3), jnp.float32),
        scratch_shapes=[pltpu.VMEM((m, k), jnp.float32)],
    )
    return jax.vmap(call, in_axes=(0, 0, 0, None, None, None))(
        xyz_t, xsrc, cent_rows, w1, w2, w3)


def _run_fp(sxyz_rows, dxyz_t, dfeat, sfeat, ws, head_ws=None):
    s, d = sxyz_rows.shape[-2], dxyz_t.shape[-1]
    nw = len(ws)
    head = head_ws is not None
    all_ws = list(ws) + (list(head_ws) if head else [])
    cout = 13 if head else ws[-1].shape[-1]
    fn = functools.partial(_fp_kernel, s, d, nw, head)
    call = pl.pallas_call(
        fn,
        out_shape=jax.ShapeDtypeStruct((s, cout), jnp.float32),
    )
    in_axes = (0, 0, 0, 0) + (None,) * len(all_ws)
    return jax.vmap(call, in_axes=in_axes)(
        sxyz_rows, dxyz_t, dfeat, sfeat, *all_ws)


def kernel(points, params):
    xyz_rows = jnp.transpose(points[:, 0:3, :], (0, 2, 1))
    xyz_t = points[:, 0:3, :]
    feat_rows = jnp.transpose(points[:, 3:, :], (0, 2, 1))

    inter_xyz_rows = [xyz_rows]
    inter_xyz_t = [xyz_t]
    inter_feat = [jnp.transpose(points, (0, 2, 1))]

    cur_xyz_t, cur_xyz_rows, cur_feat = xyz_t, xyz_rows, feat_rows
    for level in range(4):
        xsrc = jnp.concatenate([cur_xyz_rows, cur_feat], axis=-1)
        if level == 0:
            b, n, _ = xsrc.shape
            xsrc = jnp.pad(xsrc, ((0, 0), (0, 0), (0, 2)))
            xsrc = xsrc.reshape(b, 32, 128, 8).transpose(0, 2, 1, 3)
            xsrc = xsrc.reshape(b, 128, 256)
            w1, w2, w3 = params['sa'][0]
            w1 = jnp.pad(w1, ((0, 2), (0, 0)))
        else:
            w1, w2, w3 = params['sa'][level]
        cent_t, cent_rows = _run_fps(level, cur_xyz_t)
        feat = _run_sa(level, cur_xyz_t, xsrc, cent_rows, w1, w2, w3)
        cur_xyz_t, cur_xyz_rows, cur_feat = cent_t, cent_rows, feat
        inter_xyz_rows.append(cent_rows)
        inter_xyz_t.append(cent_t)
        inter_feat.append(feat)

    dfeat = jnp.concatenate([cur_xyz_rows, cur_feat], axis=-1)
    dxyz_t = cur_xyz_t
    for i in range(4):
        sxyz_rows = inter_xyz_rows[-1 - i]
        sfeat = inter_feat[-1 - i]
        head_ws = None
        if i == 3:
            head_ws = [params['seg'][0], params['logit_w'],
                       params['logit_b'].reshape(1, 13)]
        dfeat = _run_fp(sxyz_rows, dxyz_t, dfeat, sfeat, params['fp'][i],
                        head_ws)
        dxyz_t = inter_xyz_t[-1 - i]
    return jnp.transpose(dfeat, (0, 2, 1))


# speedup vs baseline: 5.9328x; 1.0025x over previous
import functools

import jax
import jax.numpy as jnp
import numpy as np
from jax import lax
from jax.experimental import pallas as pl
from jax.experimental.pallas import tpu as pltpu
from jax.experimental.pallas import tpu_sc as plsc


def _sc_gather(table, idx):
    v, d = table.shape
    bt = idx.shape[0]
    info = plsc.get_sparse_core_info()
    nw = info.num_cores * info.num_subcores
    bpw = bt // nw
    mesh = plsc.VectorSubcoreMesh(core_axis_name="c", subcore_axis_name="s")

    @functools.partial(
        pl.kernel, mesh=mesh,
        out_type=jax.ShapeDtypeStruct((bt, d), jnp.float32),
        compiler_params=pltpu.CompilerParams(use_tc_tiling_on_sc=False),
        scratch_types=[
            pltpu.VMEM((bpw,), jnp.int32),
            pltpu.VMEM((bpw, d), jnp.float32),
            pltpu.SemaphoreType.DMA,
        ],
    )
    def k(table_hbm, idx_hbm, out_hbm, idx_v, rows_v, sem):
        wid = lax.axis_index("s") * info.num_cores + lax.axis_index("c")
        base = wid * bpw
        pltpu.sync_copy(idx_hbm.at[pl.ds(base, bpw)], idx_v)
        pltpu.async_copy(table_hbm.at[idx_v], rows_v, sem).wait()
        pltpu.sync_copy(rows_v, out_hbm.at[pl.ds(base, bpw)])

    return k(table, idx)

_BN = np.float32(1.0 / np.sqrt(1.0 + 1e-5))
_SA_CFG = [(1024, 0.1, 32), (256, 0.2, 32), (64, 0.4, 32), (16, 0.8, 32)]


def _iota_f32(shape, dim):
    return jax.lax.broadcasted_iota(jnp.int32, shape, dim).astype(jnp.float32)


def _first_min_idx(vals, iota_lanes, big):
    v = jnp.min(vals, axis=1, keepdims=True)
    idx = jnp.min(jnp.where(vals <= v, iota_lanes, big), axis=1, keepdims=True)
    return v, idx


def _fps_kernel(n, m, xyz_f, cent_ref):
    x = xyz_f[...]
    x0, x1, x2 = x[0], x[1], x[2]
    r, l = x0.shape
    cr, cl = cent_ref.shape[1], cent_ref.shape[2]
    flat = (jax.lax.broadcasted_iota(jnp.int32, (r, l), 0) * l
            + jax.lax.broadcasted_iota(jnp.int32, (r, l), 1))
    cflat = (jax.lax.broadcasted_iota(jnp.int32, (cr, cl), 0) * cl
             + jax.lax.broadcasted_iota(jnp.int32, (cr, cl), 1))

    def step(t, carry):
        dists, far, cx, cy, cz = carry
        oh = (flat == far).astype(jnp.float32)
        c0 = jnp.sum(x0 * oh)
        c1 = jnp.sum(x1 * oh)
        c2 = jnp.sum(x2 * oh)
        cx = jnp.where(cflat == t, c0, cx)
        cy = jnp.where(cflat == t, c1, cy)
        cz = jnp.where(cflat == t, c2, cz)
        d = (x0 - c0) ** 2 + (x1 - c1) ** 2 + (x2 - c2) ** 2
        dists = jnp.minimum(dists, d)
        mx = jnp.max(dists)
        far2 = jnp.min(jnp.where(dists >= mx, flat, n))
        return dists, far2, cx, cy, cz

    zc = jnp.zeros((cr, cl), jnp.float32)
    init = (jnp.full((r, l), 1e10, jnp.float32), jnp.int32(0), zc, zc, zc)
    _, _, cx, cy, cz = jax.lax.fori_loop(0, m, step, init)
    cent_ref[...] = jnp.stack([cx, cy, cz], axis=0)


def _run_fps(level, xyz_t):
    b, _, n = xyz_t.shape
    m = _SA_CFG[level][0]
    l = min(n, 128)
    cl = min(m, 128)
    xyz_f = xyz_t.reshape(b, 3, n // l, l)
    call = pl.pallas_call(
        functools.partial(_fps_kernel, n, m),
        out_shape=jax.ShapeDtypeStruct((3, m // cl, cl), jnp.float32),
    )
    cent = jax.vmap(call)(xyz_f)
    cent_t = cent.reshape(b, 3, m)
    return cent_t, jnp.transpose(cent_t, (0, 2, 1))


def _ball_query_body(xyz_t, cent_rows, n, m, radius, k, mc, gidx_ref):
    x = xyz_t[...]
    xx = jnp.sum(x * x, axis=0, keepdims=True)
    lane_iota = _iota_f32((1, n), 1)
    lane_k = jax.lax.broadcasted_iota(jnp.int32, (1, k), 1)
    r2 = np.float32(radius * radius)
    nf = np.float32(n)
    for c0 in range(0, m, mc):
        cent = jax.lax.slice(cent_rows, (c0, 0), (c0 + mc, 3))
        cc = jnp.sum(cent * cent, axis=1, keepdims=True)
        ab = jax.lax.dot_general(
            cent, x, (((1,), (0,)), ((), ())),
            preferred_element_type=jnp.float32)
        d2 = jnp.maximum(cc + xx - 2.0 * ab, 0.0)
        cand = jnp.where(d2 <= r2, lane_iota, nf)
        v0 = jnp.min(cand, axis=1, keepdims=True)

        def kstep(kk, carry):
            cand, slots = carry
            v = jnp.min(cand, axis=1, keepdims=True)
            slot = jnp.where(v < nf, v, v0)
            slots = jnp.where(lane_k == kk, slot, slots)
            return jnp.where(cand == v, np.float32(1e9), cand), slots

        _, slots = jax.lax.fori_loop(
            0, k, kstep, (cand, jnp.zeros((mc, k), jnp.float32)))
        gidx_ref[pl.ds(c0, mc), :] = slots


def _bq_kernel(n, m, radius, k, mc, xyz_t, cent_rows_in, gidx_ref):
    cent_rows = cent_rows_in[...]
    _ball_query_body(xyz_t, cent_rows, n, m, radius, k, mc, gidx_ref)


def _mlp_kernel(m, k, g_tile, grouped, cent_rows_in, w1, w2, w3, feat_ref):
    w1v, w2v, w3v = w1[...], w2[...], w3[...]
    w1_3 = w1v[:3, :]
    rows = g_tile * k

    def tile(t, _):
        t0 = pl.multiple_of(t * g_tile, g_tile)
        r0 = pl.multiple_of(t * rows, rows)
        g = grouped[pl.ds(r0, rows), :]
        cent = cent_rows_in[pl.ds(t0, g_tile), :]
        cw = jnp.dot(cent, w1_3, preferred_element_type=jnp.float32)
        h = jnp.dot(g, w1v, preferred_element_type=jnp.float32)
        h = h.reshape(g_tile, k, -1) - cw[:, None, :]
        h = jax.nn.relu(h * _BN).reshape(rows, -1)
        h = jax.nn.relu(jnp.dot(h, w2v, preferred_element_type=jnp.float32) * _BN)
        h = jax.nn.relu(jnp.dot(h, w3v, preferred_element_type=jnp.float32) * _BN)
        h = jnp.max(h.reshape(g_tile, k, -1), axis=1)
        feat_ref[pl.ds(t0, g_tile), :] = h
        return 0

    jax.lax.fori_loop(0, m // g_tile, tile, 0)


def _fp_kernel(s, d, nw, head, sxyz_rows, dxyz_t, dfeat, sfeat, *refs):
    ws_refs, out_ref = refs[:-1], refs[-1]
    sx = sxyz_rows[...]
    dx = dxyz_t[...]
    cc_s = jnp.sum(sx * sx, axis=1, keepdims=True)
    cc_d = jnp.sum(dx * dx, axis=0, keepdims=True)
    ab = jax.lax.dot_general(sx, dx, (((1,), (0,)), ((), ())),
                             preferred_element_type=jnp.float32)
    d2 = jnp.maximum(cc_s + cc_d - 2.0 * ab, 0.0)
    lane_iota = _iota_f32((1, d), 1)
    big = np.float32(d)
    vs, idxs = [], []
    for _ in range(3):
        v, idx = _first_min_idx(d2, lane_iota, big)
        vs.append(v)
        idxs.append(idx)
        d2 = jnp.where(lane_iota == idx, np.float32(1e9), d2)
    ws = [1.0 / (v + np.float32(1e-8)) for v in vs]
    wsum = ws[0] + ws[1] + ws[2]
    df = dfeat[...]
    interp = None
    for v_idx, w in zip(idxs, ws):
        oh = (lane_iota == v_idx).astype(jnp.float32)
        gathered = jnp.dot(oh, df, preferred_element_type=jnp.float32)
        term = gathered * (w / wsum)
        interp = term if interp is None else interp + term
    f = jnp.concatenate([interp, sfeat[...]], axis=1)
    for i in range(nw):
        wv = ws_refs[i][...]
        f = jax.nn.relu(jnp.dot(f, wv, preferred_element_type=jnp.float32) * _BN)
    if head:
        seg_w, logit_w, logit_b = ws_refs[nw][...], ws_refs[nw + 1][...], ws_refs[nw + 2][...]
        f = jax.nn.relu(jnp.dot(f, seg_w, preferred_element_type=jnp.float32) * _BN)
        f = jnp.dot(f, logit_w, preferred_element_type=jnp.float32) + logit_b
    out_ref[...] = f


def _run_sa(level, xyz_t, xsrc_rows, cent_rows, w1, w2, w3):
    m, radius, k, n = *_SA_CFG[level], xyz_t.shape[-1]
    b = xyz_t.shape[0]
    g_tile = 64 if level == 0 else 16
    mc = min(m, 256)
    c3 = w3.shape[-1]

    bq = pl.pallas_call(
        functools.partial(_bq_kernel, n, m, radius, k, mc),
        out_shape=jax.ShapeDtypeStruct((m, k), jnp.float32),
    )
    gidx = jax.vmap(bq)(xyz_t, cent_rows)

    cs = xsrc_rows.shape[-1]
    cp = ((cs + 15) // 16) * 16
    table = jnp.pad(xsrc_rows, ((0, 0), (0, 0), (0, cp - cs)))
    table = table.reshape(b * n, cp)
    idx = (gidx.reshape(b, m * k).astype(jnp.int32)
           + (jnp.arange(b, dtype=jnp.int32) * n)[:, None]).reshape(-1)
    grouped = _sc_gather(table, idx).reshape(b, m * k, cp)

    w1p = jnp.pad(w1, ((0, cp - cs), (0, 0)))
    mlp = pl.pallas_call(
        functools.partial(_mlp_kernel, m, k, g_tile),
        out_shape=jax.ShapeDtypeStruct((m, c3), jnp.float32),
    )
    return jax.vmap(mlp, in_axes=(0, 0, None, None, None))(
        grouped, cent_rows, w1p, w2, w3)


def _run_fp(sxyz_rows, dxyz_t, dfeat, sfeat, ws, head_ws=None):
    s, d = sxyz_rows.shape[-2], dxyz_t.shape[-1]
    nw = len(ws)
    head = head_ws is not None
    all_ws = list(ws) + (list(head_ws) if head else [])
    cout = 13 if head else ws[-1].shape[-1]
    fn = functools.partial(_fp_kernel, s, d, nw, head)
    call = pl.pallas_call(
        fn,
        out_shape=jax.ShapeDtypeStruct((s, cout), jnp.float32),
    )
    in_axes = (0, 0, 0, 0) + (None,) * len(all_ws)
    return jax.vmap(call, in_axes=in_axes)(
        sxyz_rows, dxyz_t, dfeat, sfeat, *all_ws)


def kernel(points, params):
    xyz_rows = jnp.transpose(points[:, 0:3, :], (0, 2, 1))
    xyz_t = points[:, 0:3, :]
    feat_rows = jnp.transpose(points[:, 3:, :], (0, 2, 1))

    inter_xyz_rows = [xyz_rows]
    inter_xyz_t = [xyz_t]
    inter_feat = [jnp.transpose(points, (0, 2, 1))]

    cur_xyz_t, cur_xyz_rows, cur_feat = xyz_t, xyz_rows, feat_rows
    for level in range(4):
        xsrc_rows = jnp.concatenate([cur_xyz_rows, cur_feat], axis=-1)
        w1, w2, w3 = params['sa'][level]
        cent_t, cent_rows = _run_fps(level, cur_xyz_t)
        feat = _run_sa(level, cur_xyz_t, xsrc_rows, cent_rows, w1, w2, w3)
        cur_xyz_t, cur_xyz_rows, cur_feat = cent_t, cent_rows, feat
        inter_xyz_rows.append(cent_rows)
        inter_xyz_t.append(cent_t)
        inter_feat.append(feat)

    dfeat = jnp.concatenate([cur_xyz_rows, cur_feat], axis=-1)
    dxyz_t = cur_xyz_t
    for i in range(4):
        sxyz_rows = inter_xyz_rows[-1 - i]
        sfeat = inter_feat[-1 - i]
        head_ws = None
        if i == 3:
            head_ws = [params['seg'][0], params['logit_w'],
                       params['logit_b'].reshape(1, 13)]
        dfeat = _run_fp(sxyz_rows, dxyz_t, dfeat, sfeat, params['fp'][i],
                        head_ws)
        dxyz_t = inter_xyz_t[-1 - i]
    return jnp.transpose(dfeat, (0, 2, 1))


# speedup vs baseline: 14.8638x; 2.5054x over previous
import functools

import jax
import jax.numpy as jnp
import numpy as np
from jax import lax
from jax.experimental import pallas as pl
from jax.experimental.pallas import tpu as pltpu
from jax.experimental.pallas import tpu_sc as plsc


def _sc_gather(table, idx):
    v, d = table.shape
    bt = idx.shape[0]
    info = plsc.get_sparse_core_info()
    nw = info.num_cores * info.num_subcores
    bpw = bt // nw
    mesh = plsc.VectorSubcoreMesh(core_axis_name="c", subcore_axis_name="s")

    @functools.partial(
        pl.kernel, mesh=mesh,
        out_type=jax.ShapeDtypeStruct((bt, d), jnp.float32),
        compiler_params=pltpu.CompilerParams(use_tc_tiling_on_sc=False),
        scratch_types=[
            pltpu.VMEM((bpw,), jnp.int32),
            pltpu.VMEM((bpw, d), jnp.float32),
            pltpu.SemaphoreType.DMA,
        ],
    )
    def k(table_hbm, idx_hbm, out_hbm, idx_v, rows_v, sem):
        wid = lax.axis_index("s") * info.num_cores + lax.axis_index("c")
        base = wid * bpw
        pltpu.sync_copy(idx_hbm.at[pl.ds(base, bpw)], idx_v)
        pltpu.async_copy(table_hbm.at[idx_v], rows_v, sem).wait()
        pltpu.sync_copy(rows_v, out_hbm.at[pl.ds(base, bpw)])

    return k(table, idx)

_BN = np.float32(1.0 / np.sqrt(1.0 + 1e-5))
_SA_CFG = [(1024, 0.1, 32), (256, 0.2, 32), (64, 0.4, 32), (16, 0.8, 32)]


def _iota_f32(shape, dim):
    return jax.lax.broadcasted_iota(jnp.int32, shape, dim).astype(jnp.float32)


def _first_min_idx(vals, iota_lanes, big):
    v = jnp.min(vals, axis=1, keepdims=True)
    idx = jnp.min(jnp.where(vals <= v, iota_lanes, big), axis=1, keepdims=True)
    return v, idx


def _fps_kernel(n, m, xyz_f, cent_ref):
    x = xyz_f[...]
    x0, x1, x2 = x[:, 0], x[:, 1], x[:, 2]
    b, r, l = x0.shape
    cr, cl = cent_ref.shape[2], cent_ref.shape[3]
    flat = (jax.lax.broadcasted_iota(jnp.int32, (b, r, l), 1) * l
            + jax.lax.broadcasted_iota(jnp.int32, (b, r, l), 2))
    cflat = (jax.lax.broadcasted_iota(jnp.int32, (b, cr, cl), 1) * cl
             + jax.lax.broadcasted_iota(jnp.int32, (b, cr, cl), 2))

    def red2(op, a):
        return op(op(a, axis=2), axis=1)[:, None, None]

    def step(t, carry):
        dists, far, cx, cy, cz = carry
        oh = (flat == far).astype(jnp.float32)
        c0 = red2(jnp.sum, x0 * oh)
        c1 = red2(jnp.sum, x1 * oh)
        c2 = red2(jnp.sum, x2 * oh)
        sel = cflat == t
        cx = jnp.where(sel, c0, cx)
        cy = jnp.where(sel, c1, cy)
        cz = jnp.where(sel, c2, cz)
        d = (x0 - c0) ** 2 + (x1 - c1) ** 2 + (x2 - c2) ** 2
        dists = jnp.minimum(dists, d)
        mx = red2(jnp.max, dists)
        far2 = red2(jnp.min, jnp.where(dists >= mx, flat, n))
        return dists, far2, cx, cy, cz

    zc = jnp.zeros((b, cr, cl), jnp.float32)
    init = (jnp.full((b, r, l), 1e10, jnp.float32),
            jnp.zeros((b, 1, 1), jnp.int32), zc, zc, zc)
    _, _, cx, cy, cz = jax.lax.fori_loop(0, m, step, init)
    cent_ref[...] = jnp.stack([cx, cy, cz], axis=1)


def _run_fps(level, xyz_t):
    b, _, n = xyz_t.shape
    m = _SA_CFG[level][0]
    l = min(n, 128)
    cl = min(m, 128)
    xyz_f = xyz_t.reshape(b, 3, n // l, l)
    call = pl.pallas_call(
        functools.partial(_fps_kernel, n, m),
        out_shape=jax.ShapeDtypeStruct((b, 3, m // cl, cl), jnp.float32),
    )
    cent = call(xyz_f)
    cent_t = cent.reshape(b, 3, m)
    return cent_t, jnp.transpose(cent_t, (0, 2, 1))


def _ball_query_body(xyz_t, cent_rows, n, m, radius, k, mc, gidx_ref):
    x = xyz_t[...]
    xx = jnp.sum(x * x, axis=0, keepdims=True)
    lane_iota = _iota_f32((1, n), 1)
    lane_k = jax.lax.broadcasted_iota(jnp.int32, (1, k), 1)
    r2 = np.float32(radius * radius)
    nf = np.float32(n)
    for c0 in range(0, m, mc):
        cent = jax.lax.slice(cent_rows, (c0, 0), (c0 + mc, 3))
        cc = jnp.sum(cent * cent, axis=1, keepdims=True)
        ab = jax.lax.dot_general(
            cent, x, (((1,), (0,)), ((), ())),
            preferred_element_type=jnp.float32)
        d2 = jnp.maximum(cc + xx - 2.0 * ab, 0.0)
        cand = jnp.where(d2 <= r2, lane_iota, nf)
        v0 = jnp.min(cand, axis=1, keepdims=True)

        def cond(carry):
            kk, go, _, _ = carry
            return (kk < k) & go

        def kstep(carry):
            kk, _, cand, slots = carry
            v = jnp.min(cand, axis=1, keepdims=True)
            slots = jnp.where((lane_k == kk) & (v < nf), v, slots)
            cand = jnp.where(cand == v, np.float32(1e9), cand)
            return kk + 1, jnp.min(v) < nf, cand, slots

        slots0 = jnp.zeros((mc, k), jnp.float32) + v0
        _, _, _, slots = jax.lax.while_loop(
            cond, kstep, (jnp.int32(0), jnp.bool_(True), cand, slots0))
        gidx_ref[pl.ds(c0, mc), :] = slots


def _bq_kernel(n, m, radius, k, mc, xyz_t, cent_rows_in, gidx_ref):
    cent_rows = cent_rows_in[...]
    _ball_query_body(xyz_t, cent_rows, n, m, radius, k, mc, gidx_ref)


def _mlp_kernel(m, k, g_tile, grouped, cent_rows_in, w1, w2, w3, feat_ref):
    w1v, w2v, w3v = w1[...], w2[...], w3[...]
    w1_3 = w1v[:3, :]
    rows = g_tile * k

    def tile(t, _):
        t0 = pl.multiple_of(t * g_tile, g_tile)
        r0 = pl.multiple_of(t * rows, rows)
        g = grouped[pl.ds(r0, rows), :]
        cent = cent_rows_in[pl.ds(t0, g_tile), :]
        cw = jnp.dot(cent, w1_3, preferred_element_type=jnp.float32)
        h = jnp.dot(g, w1v, preferred_element_type=jnp.float32)
        h = h.reshape(g_tile, k, -1) - cw[:, None, :]
        h = jax.nn.relu(h * _BN).reshape(rows, -1)
        h = jax.nn.relu(jnp.dot(h, w2v, preferred_element_type=jnp.float32) * _BN)
        h = jax.nn.relu(jnp.dot(h, w3v, preferred_element_type=jnp.float32) * _BN)
        h = jnp.max(h.reshape(g_tile, k, -1), axis=1)
        feat_ref[pl.ds(t0, g_tile), :] = h
        return 0

    jax.lax.fori_loop(0, m // g_tile, tile, 0)


def _fp_kernel(s, d, nw, head, sxyz_rows, dxyz_t, dfeat, sfeat, *refs):
    ws_refs, out_ref = refs[:-1], refs[-1]
    sx = sxyz_rows[...]
    dx = dxyz_t[...]
    cc_s = jnp.sum(sx * sx, axis=1, keepdims=True)
    cc_d = jnp.sum(dx * dx, axis=0, keepdims=True)
    ab = jax.lax.dot_general(sx, dx, (((1,), (0,)), ((), ())),
                             preferred_element_type=jnp.float32)
    d2 = jnp.maximum(cc_s + cc_d - 2.0 * ab, 0.0)
    lane_iota = _iota_f32((1, d), 1)
    big = np.float32(d)
    vs, idxs = [], []
    for _ in range(3):
        v, idx = _first_min_idx(d2, lane_iota, big)
        vs.append(v)
        idxs.append(idx)
        d2 = jnp.where(lane_iota == idx, np.float32(1e9), d2)
    ws = [1.0 / (v + np.float32(1e-8)) for v in vs]
    wsum = ws[0] + ws[1] + ws[2]
    df = dfeat[...]
    interp = None
    for v_idx, w in zip(idxs, ws):
        oh = (lane_iota == v_idx).astype(jnp.float32)
        gathered = jnp.dot(oh, df, preferred_element_type=jnp.float32)
        term = gathered * (w / wsum)
        interp = term if interp is None else interp + term
    f = jnp.concatenate([interp, sfeat[...]], axis=1)
    for i in range(nw):
        wv = ws_refs[i][...]
        f = jax.nn.relu(jnp.dot(f, wv, preferred_element_type=jnp.float32) * _BN)
    if head:
        seg_w, logit_w, logit_b = ws_refs[nw][...], ws_refs[nw + 1][...], ws_refs[nw + 2][...]
        f = jax.nn.relu(jnp.dot(f, seg_w, preferred_element_type=jnp.float32) * _BN)
        f = jnp.dot(f, logit_w, preferred_element_type=jnp.float32) + logit_b
    out_ref[...] = f


def _run_sa(level, xyz_t, xsrc_rows, cent_rows, w1, w2, w3):
    m, radius, k, n = *_SA_CFG[level], xyz_t.shape[-1]
    b = xyz_t.shape[0]
    g_tile = 64 if level == 0 else 16
    mc = min(m, 256)
    c3 = w3.shape[-1]

    bq = pl.pallas_call(
        functools.partial(_bq_kernel, n, m, radius, k, mc),
        out_shape=jax.ShapeDtypeStruct((m, k), jnp.float32),
    )
    gidx = jax.vmap(bq)(xyz_t, cent_rows)

    cs = xsrc_rows.shape[-1]
    cp = ((cs + 15) // 16) * 16
    table = jnp.pad(xsrc_rows, ((0, 0), (0, 0), (0, cp - cs)))
    table = table.reshape(b * n, cp)
    idx = (gidx.reshape(b, m * k).astype(jnp.int32)
           + (jnp.arange(b, dtype=jnp.int32) * n)[:, None]).reshape(-1)
    grouped = _sc_gather(table, idx).reshape(b, m * k, cp)

    w1p = jnp.pad(w1, ((0, cp - cs), (0, 0)))
    mlp = pl.pallas_call(
        functools.partial(_mlp_kernel, m, k, g_tile),
        out_shape=jax.ShapeDtypeStruct((m, c3), jnp.float32),
    )
    return jax.vmap(mlp, in_axes=(0, 0, None, None, None))(
        grouped, cent_rows, w1p, w2, w3)


def _run_fp(sxyz_rows, dxyz_t, dfeat, sfeat, ws, head_ws=None):
    s, d = sxyz_rows.shape[-2], dxyz_t.shape[-1]
    nw = len(ws)
    head = head_ws is not None
    all_ws = list(ws) + (list(head_ws) if head else [])
    cout = 13 if head else ws[-1].shape[-1]
    fn = functools.partial(_fp_kernel, s, d, nw, head)
    call = pl.pallas_call(
        fn,
        out_shape=jax.ShapeDtypeStruct((s, cout), jnp.float32),
    )
    in_axes = (0, 0, 0, 0) + (None,) * len(all_ws)
    return jax.vmap(call, in_axes=in_axes)(
        sxyz_rows, dxyz_t, dfeat, sfeat, *all_ws)


def kernel(points, params):
    xyz_rows = jnp.transpose(points[:, 0:3, :], (0, 2, 1))
    xyz_t = points[:, 0:3, :]
    feat_rows = jnp.transpose(points[:, 3:, :], (0, 2, 1))

    inter_xyz_rows = [xyz_rows]
    inter_xyz_t = [xyz_t]
    inter_feat = [jnp.transpose(points, (0, 2, 1))]

    cur_xyz_t, cur_xyz_rows, cur_feat = xyz_t, xyz_rows, feat_rows
    for level in range(4):
        xsrc_rows = jnp.concatenate([cur_xyz_rows, cur_feat], axis=-1)
        w1, w2, w3 = params['sa'][level]
        cent_t, cent_rows = _run_fps(level, cur_xyz_t)
        feat = _run_sa(level, cur_xyz_t, xsrc_rows, cent_rows, w1, w2, w3)
        cur_xyz_t, cur_xyz_rows, cur_feat = cent_t, cent_rows, feat
        inter_xyz_rows.append(cent_rows)
        inter_xyz_t.append(cent_t)
        inter_feat.append(feat)

    dfeat = jnp.concatenate([cur_xyz_rows, cur_feat], axis=-1)
    dxyz_t = cur_xyz_t
    for i in range(4):
        sxyz_rows = inter_xyz_rows[-1 - i]
        sfeat = inter_feat[-1 - i]
        head_ws = None
        if i == 3:
            head_ws = [params['seg'][0], params['logit_w'],
                       params['logit_b'].reshape(1, 13)]
        dfeat = _run_fp(sxyz_rows, dxyz_t, dfeat, sfeat, params['fp'][i],
                        head_ws)
        dxyz_t = inter_xyz_t[-1 - i]
    return jnp.transpose(dfeat, (0, 2, 1))


# speedup vs baseline: 15.0936x; 1.0155x over previous
import functools

import jax
import jax.numpy as jnp
import numpy as np
from jax import lax
from jax.experimental import pallas as pl
from jax.experimental.pallas import tpu as pltpu
from jax.experimental.pallas import tpu_sc as plsc


def _sc_gather(table, idx):
    v, d = table.shape
    bt = idx.shape[0]
    info = plsc.get_sparse_core_info()
    nw = info.num_cores * info.num_subcores
    bpw = bt // nw
    mesh = plsc.VectorSubcoreMesh(core_axis_name="c", subcore_axis_name="s")

    @functools.partial(
        pl.kernel, mesh=mesh,
        out_type=jax.ShapeDtypeStruct((bt, d), jnp.float32),
        compiler_params=pltpu.CompilerParams(use_tc_tiling_on_sc=False),
        scratch_types=[
            pltpu.VMEM((bpw,), jnp.int32),
            pltpu.VMEM((bpw, d), jnp.float32),
            pltpu.SemaphoreType.DMA,
        ],
    )
    def k(table_hbm, idx_hbm, out_hbm, idx_v, rows_v, sem):
        wid = lax.axis_index("s") * info.num_cores + lax.axis_index("c")
        base = wid * bpw
        pltpu.sync_copy(idx_hbm.at[pl.ds(base, bpw)], idx_v)
        pltpu.async_copy(table_hbm.at[idx_v], rows_v, sem).wait()
        pltpu.sync_copy(rows_v, out_hbm.at[pl.ds(base, bpw)])

    return k(table, idx)

_BN = np.float32(1.0 / np.sqrt(1.0 + 1e-5))
_SA_CFG = [(1024, 0.1, 32), (256, 0.2, 32), (64, 0.4, 32), (16, 0.8, 32)]


def _iota_f32(shape, dim):
    return jax.lax.broadcasted_iota(jnp.int32, shape, dim).astype(jnp.float32)


def _first_min_idx(vals, iota_lanes, big):
    v = jnp.min(vals, axis=1, keepdims=True)
    idx = jnp.min(jnp.where(vals <= v, iota_lanes, big), axis=1, keepdims=True)
    return v, idx


def _packed_argmax(d, idx, xs):
    for axis in (1, 2):
        while d.shape[axis] > 1:
            h = d.shape[axis] // 2
            lo = lambda a: lax.slice_in_dim(a, 0, h, axis=axis)
            hi = lambda a: lax.slice_in_dim(a, h, 2 * h, axis=axis)
            d_lo, d_hi = lo(d), hi(d)
            i_lo, i_hi = lo(idx), hi(idx)
            take = (d_hi > d_lo) | ((d_hi == d_lo) & (i_hi < i_lo))
            d = jnp.where(take, d_hi, d_lo)
            idx = jnp.where(take, i_hi, i_lo)
            xs = [jnp.where(take, hi(a), lo(a)) for a in xs]
    return d, idx, xs


def _fps_kernel(n, m, xyz_f, cent_ref):
    x = xyz_f[...]
    x0, x1, x2 = x[:, 0], x[:, 1], x[:, 2]
    b, r, l = x0.shape
    cr, cl = cent_ref.shape[2], cent_ref.shape[3]
    flat_f = (_iota_f32((b, r, l), 1) * l + _iota_f32((b, r, l), 2))
    cflat = (jax.lax.broadcasted_iota(jnp.int32, (b, cr, cl), 1) * cl
             + jax.lax.broadcasted_iota(jnp.int32, (b, cr, cl), 2))

    def step(t, carry):
        dists, c0, c1, c2, cx, cy, cz = carry
        sel = cflat == t
        cx = jnp.where(sel, c0, cx)
        cy = jnp.where(sel, c1, cy)
        cz = jnp.where(sel, c2, cz)
        d = (x0 - c0) ** 2 + (x1 - c1) ** 2 + (x2 - c2) ** 2
        dists = jnp.minimum(dists, d)
        _, _, (n0, n1, n2) = _packed_argmax(dists, flat_f, [x0, x1, x2])
        return dists, n0, n1, n2, cx, cy, cz

    zc = jnp.zeros((b, cr, cl), jnp.float32)
    p0 = lambda a: a[:, 0:1, 0:1]
    init = (jnp.full((b, r, l), 1e10, jnp.float32),
            p0(x0), p0(x1), p0(x2), zc, zc, zc)
    _, _, _, _, cx, cy, cz = jax.lax.fori_loop(0, m, step, init)
    cent_ref[...] = jnp.stack([cx, cy, cz], axis=1)


def _run_fps(level, xyz_t):
    b, _, n = xyz_t.shape
    m = _SA_CFG[level][0]
    l = min(n, 128)
    cl = min(m, 128)
    xyz_f = xyz_t.reshape(b, 3, n // l, l)
    call = pl.pallas_call(
        functools.partial(_fps_kernel, n, m),
        out_shape=jax.ShapeDtypeStruct((b, 3, m // cl, cl), jnp.float32),
    )
    cent = call(xyz_f)
    cent_t = cent.reshape(b, 3, m)
    return cent_t, jnp.transpose(cent_t, (0, 2, 1))


def _ball_query_body(xyz_t, cent_rows, n, m, radius, k, mc, gidx_ref):
    x = xyz_t[...]
    xx = jnp.sum(x * x, axis=0, keepdims=True)
    lane_iota = _iota_f32((1, n), 1)
    lane_k = jax.lax.broadcasted_iota(jnp.int32, (1, k), 1)
    r2 = np.float32(radius * radius)
    nf = np.float32(n)
    for c0 in range(0, m, mc):
        cent = jax.lax.slice(cent_rows, (c0, 0), (c0 + mc, 3))
        cc = jnp.sum(cent * cent, axis=1, keepdims=True)
        ab = jax.lax.dot_general(
            cent, x, (((1,), (0,)), ((), ())),
            preferred_element_type=jnp.float32)
        d2 = jnp.maximum(cc + xx - 2.0 * ab, 0.0)
        cand = jnp.where(d2 <= r2, lane_iota, nf)
        v0 = jnp.min(cand, axis=1, keepdims=True)

        def cond(carry):
            kk, go, _, _ = carry
            return (kk < k) & go

        def kstep(carry):
            kk, _, cand, slots = carry
            v = jnp.min(cand, axis=1, keepdims=True)
            slots = jnp.where((lane_k == kk) & (v < nf), v, slots)
            cand = jnp.where(cand == v, np.float32(1e9), cand)
            return kk + 1, jnp.min(v) < nf, cand, slots

        slots0 = jnp.zeros((mc, k), jnp.float32) + v0
        _, _, _, slots = jax.lax.while_loop(
            cond, kstep, (jnp.int32(0), jnp.bool_(True), cand, slots0))
        gidx_ref[pl.ds(c0, mc), :] = slots


def _bq_kernel(n, m, radius, k, mc, xyz_t, cent_rows_in, gidx_ref):
    cent_rows = cent_rows_in[...]
    _ball_query_body(xyz_t, cent_rows, n, m, radius, k, mc, gidx_ref)


def _mlp_kernel(m, k, g_tile, grouped, cent_rows_in, w1, w2, w3, feat_ref):
    w1v, w2v, w3v = w1[...], w2[...], w3[...]
    w1_3 = w1v[:3, :]
    rows = g_tile * k

    def tile(t, _):
        t0 = pl.multiple_of(t * g_tile, g_tile)
        r0 = pl.multiple_of(t * rows, rows)
        g = grouped[pl.ds(r0, rows), :]
        cent = cent_rows_in[pl.ds(t0, g_tile), :]
        cw = jnp.dot(cent, w1_3, preferred_element_type=jnp.float32)
        h = jnp.dot(g, w1v, preferred_element_type=jnp.float32)
        h = h.reshape(g_tile, k, -1) - cw[:, None, :]
        h = jax.nn.relu(h * _BN).reshape(rows, -1)
        h = jax.nn.relu(jnp.dot(h, w2v, preferred_element_type=jnp.float32) * _BN)
        h = jax.nn.relu(jnp.dot(h, w3v, preferred_element_type=jnp.float32) * _BN)
        h = jnp.max(h.reshape(g_tile, k, -1), axis=1)
        feat_ref[pl.ds(t0, g_tile), :] = h
        return 0

    jax.lax.fori_loop(0, m // g_tile, tile, 0)


def _fp_kernel(s, d, nw, head, sxyz_rows, dxyz_t, dfeat, sfeat, *refs):
    ws_refs, out_ref = refs[:-1], refs[-1]
    sx = sxyz_rows[...]
    dx = dxyz_t[...]
    cc_s = jnp.sum(sx * sx, axis=1, keepdims=True)
    cc_d = jnp.sum(dx * dx, axis=0, keepdims=True)
    ab = jax.lax.dot_general(sx, dx, (((1,), (0,)), ((), ())),
                             preferred_element_type=jnp.float32)
    d2 = jnp.maximum(cc_s + cc_d - 2.0 * ab, 0.0)
    lane_iota = _iota_f32((1, d), 1)
    big = np.float32(d)
    vs, idxs = [], []
    for _ in range(3):
        v, idx = _first_min_idx(d2, lane_iota, big)
        vs.append(v)
        idxs.append(idx)
        d2 = jnp.where(lane_iota == idx, np.float32(1e9), d2)
    ws = [1.0 / (v + np.float32(1e-8)) for v in vs]
    wsum = ws[0] + ws[1] + ws[2]
    df = dfeat[...]
    interp = None
    for v_idx, w in zip(idxs, ws):
        oh = (lane_iota == v_idx).astype(jnp.float32)
        gathered = jnp.dot(oh, df, preferred_element_type=jnp.float32)
        term = gathered * (w / wsum)
        interp = term if interp is None else interp + term
    f = jnp.concatenate([interp, sfeat[...]], axis=1)
    for i in range(nw):
        wv = ws_refs[i][...]
        f = jax.nn.relu(jnp.dot(f, wv, preferred_element_type=jnp.float32) * _BN)
    if head:
        seg_w, logit_w, logit_b = ws_refs[nw][...], ws_refs[nw + 1][...], ws_refs[nw + 2][...]
        f = jax.nn.relu(jnp.dot(f, seg_w, preferred_element_type=jnp.float32) * _BN)
        f = jnp.dot(f, logit_w, preferred_element_type=jnp.float32) + logit_b
    out_ref[...] = f


def _run_sa(level, xyz_t, xsrc_rows, cent_rows, w1, w2, w3):
    m, radius, k, n = *_SA_CFG[level], xyz_t.shape[-1]
    b = xyz_t.shape[0]
    g_tile = 64 if level == 0 else 16
    mc = min(m, 256)
    c3 = w3.shape[-1]

    bq = pl.pallas_call(
        functools.partial(_bq_kernel, n, m, radius, k, mc),
        out_shape=jax.ShapeDtypeStruct((m, k), jnp.float32),
    )
    gidx = jax.vmap(bq)(xyz_t, cent_rows)

    cs = xsrc_rows.shape[-1]
    cp = ((cs + 15) // 16) * 16
    table = jnp.pad(xsrc_rows, ((0, 0), (0, 0), (0, cp - cs)))
    table = table.reshape(b * n, cp)
    idx = (gidx.reshape(b, m * k).astype(jnp.int32)
           + (jnp.arange(b, dtype=jnp.int32) * n)[:, None]).reshape(-1)
    grouped = _sc_gather(table, idx).reshape(b, m * k, cp)

    w1p = jnp.pad(w1, ((0, cp - cs), (0, 0)))
    mlp = pl.pallas_call(
        functools.partial(_mlp_kernel, m, k, g_tile),
        out_shape=jax.ShapeDtypeStruct((m, c3), jnp.float32),
    )
    return jax.vmap(mlp, in_axes=(0, 0, None, None, None))(
        grouped, cent_rows, w1p, w2, w3)


def _run_fp(sxyz_rows, dxyz_t, dfeat, sfeat, ws, head_ws=None):
    s, d = sxyz_rows.shape[-2], dxyz_t.shape[-1]
    nw = len(ws)
    head = head_ws is not None
    all_ws = list(ws) + (list(head_ws) if head else [])
    cout = 13 if head else ws[-1].shape[-1]
    fn = functools.partial(_fp_kernel, s, d, nw, head)
    call = pl.pallas_call(
        fn,
        out_shape=jax.ShapeDtypeStruct((s, cout), jnp.float32),
    )
    in_axes = (0, 0, 0, 0) + (None,) * len(all_ws)
    return jax.vmap(call, in_axes=in_axes)(
        sxyz_rows, dxyz_t, dfeat, sfeat, *all_ws)


def kernel(points, params):
    xyz_rows = jnp.transpose(points[:, 0:3, :], (0, 2, 1))
    xyz_t = points[:, 0:3, :]
    feat_rows = jnp.transpose(points[:, 3:, :], (0, 2, 1))

    inter_xyz_rows = [xyz_rows]
    inter_xyz_t = [xyz_t]
    inter_feat = [jnp.transpose(points, (0, 2, 1))]

    cur_xyz_t, cur_xyz_rows, cur_feat = xyz_t, xyz_rows, feat_rows
    for level in range(4):
        xsrc_rows = jnp.concatenate([cur_xyz_rows, cur_feat], axis=-1)
        w1, w2, w3 = params['sa'][level]
        cent_t, cent_rows = _run_fps(level, cur_xyz_t)
        feat = _run_sa(level, cur_xyz_t, xsrc_rows, cent_rows, w1, w2, w3)
        cur_xyz_t, cur_xyz_rows, cur_feat = cent_t, cent_rows, feat
        inter_xyz_rows.append(cent_rows)
        inter_xyz_t.append(cent_t)
        inter_feat.append(feat)

    dfeat = jnp.concatenate([cur_xyz_rows, cur_feat], axis=-1)
    dxyz_t = cur_xyz_t
    for i in range(4):
        sxyz_rows = inter_xyz_rows[-1 - i]
        sfeat = inter_feat[-1 - i]
        head_ws = None
        if i == 3:
            head_ws = [params['seg'][0], params['logit_w'],
                       params['logit_b'].reshape(1, 13)]
        dfeat = _run_fp(sxyz_rows, dxyz_t, dfeat, sfeat, params['fp'][i],
                        head_ws)
        dxyz_t = inter_xyz_t[-1 - i]
    return jnp.transpose(dfeat, (0, 2, 1))


# speedup vs baseline: 15.9076x; 1.0539x over previous
import functools

import jax
import jax.numpy as jnp
import numpy as np
from jax import lax
from jax.experimental import pallas as pl
from jax.experimental.pallas import tpu as pltpu
from jax.experimental.pallas import tpu_sc as plsc


def _sc_gather(table, idx):
    v, d = table.shape
    bt = idx.shape[0]
    info = plsc.get_sparse_core_info()
    nw = info.num_cores * info.num_subcores
    bpw = bt // nw
    mesh = plsc.VectorSubcoreMesh(core_axis_name="c", subcore_axis_name="s")

    @functools.partial(
        pl.kernel, mesh=mesh,
        out_type=jax.ShapeDtypeStruct((bt, d), jnp.float32),
        compiler_params=pltpu.CompilerParams(use_tc_tiling_on_sc=False),
        scratch_types=[
            pltpu.VMEM((bpw,), jnp.int32),
            pltpu.VMEM((bpw, d), jnp.float32),
            pltpu.SemaphoreType.DMA,
        ],
    )
    def k(table_hbm, idx_hbm, out_hbm, idx_v, rows_v, sem):
        wid = lax.axis_index("s") * info.num_cores + lax.axis_index("c")
        base = wid * bpw
        pltpu.sync_copy(idx_hbm.at[pl.ds(base, bpw)], idx_v)
        pltpu.async_copy(table_hbm.at[idx_v], rows_v, sem).wait()
        pltpu.sync_copy(rows_v, out_hbm.at[pl.ds(base, bpw)])

    return k(table, idx)

_BN = np.float32(1.0 / np.sqrt(1.0 + 1e-5))
_SA_CFG = [(1024, 0.1, 32), (256, 0.2, 32), (64, 0.4, 32), (16, 0.8, 32)]


def _iota_f32(shape, dim):
    return jax.lax.broadcasted_iota(jnp.int32, shape, dim).astype(jnp.float32)


def _first_min_idx(vals, iota_lanes, big):
    v = jnp.min(vals, axis=1, keepdims=True)
    idx = jnp.min(jnp.where(vals <= v, iota_lanes, big), axis=1, keepdims=True)
    return v, idx


def _packed_argmax(d, idx, xs):
    for axis in (1, 2):
        while d.shape[axis] > 1:
            h = d.shape[axis] // 2
            lo = lambda a: lax.slice_in_dim(a, 0, h, axis=axis)
            hi = lambda a: lax.slice_in_dim(a, h, 2 * h, axis=axis)
            d_lo, d_hi = lo(d), hi(d)
            i_lo, i_hi = lo(idx), hi(idx)
            take = (d_hi > d_lo) | ((d_hi == d_lo) & (i_hi < i_lo))
            d = jnp.where(take, d_hi, d_lo)
            idx = jnp.where(take, i_hi, i_lo)
            xs = [jnp.where(take, hi(a), lo(a)) for a in xs]
    return d, idx, xs


def _fps_kernel(n, m, xyz_f, cent_ref):
    x = xyz_f[...]
    x0, x1, x2 = x[:, 0], x[:, 1], x[:, 2]
    b, r, l = x0.shape
    cr, cl = cent_ref.shape[2], cent_ref.shape[3]
    flat_f = (_iota_f32((b, r, l), 1) * l + _iota_f32((b, r, l), 2))
    cflat = (jax.lax.broadcasted_iota(jnp.int32, (b, cr, cl), 1) * cl
             + jax.lax.broadcasted_iota(jnp.int32, (b, cr, cl), 2))

    def step(t, carry):
        dists, c0, c1, c2, cx, cy, cz = carry
        sel = cflat == t
        cx = jnp.where(sel, c0, cx)
        cy = jnp.where(sel, c1, cy)
        cz = jnp.where(sel, c2, cz)
        d = (x0 - c0) ** 2 + (x1 - c1) ** 2 + (x2 - c2) ** 2
        dists = jnp.minimum(dists, d)
        _, _, (n0, n1, n2) = _packed_argmax(dists, flat_f, [x0, x1, x2])
        return dists, n0, n1, n2, cx, cy, cz

    zc = jnp.zeros((b, cr, cl), jnp.float32)
    p0 = lambda a: a[:, 0:1, 0:1]
    init = (jnp.full((b, r, l), 1e10, jnp.float32),
            p0(x0), p0(x1), p0(x2), zc, zc, zc)
    _, _, _, _, cx, cy, cz = jax.lax.fori_loop(0, m, step, init)
    cent_ref[...] = jnp.stack([cx, cy, cz], axis=1)


def _run_fps(level, xyz_t):
    b, _, n = xyz_t.shape
    m = _SA_CFG[level][0]
    l = min(n, 128)
    cl = min(m, 128)
    xyz_f = xyz_t.reshape(b, 3, n // l, l)
    call = pl.pallas_call(
        functools.partial(_fps_kernel, n, m),
        out_shape=jax.ShapeDtypeStruct((b, 3, m // cl, cl), jnp.float32),
    )
    cent = call(xyz_f)
    cent_t = cent.reshape(b, 3, m)
    return cent_t, jnp.transpose(cent_t, (0, 2, 1))


def _ball_query_body(xyz_t, cent_rows, plo, phi, n, m, radius, k, mc,
                     gidx_ref):
    x = xyz_t[...]
    xx = jnp.sum(x * x, axis=0, keepdims=True)
    plo_v, phi_v = plo[...], phi[...]
    w_cnt = plo_v.shape[1]
    lane_k = jax.lax.broadcasted_iota(jnp.int32, (1, k), 1)
    lane_w = jax.lax.broadcasted_iota(jnp.int32, (1, w_cnt), 1)
    wbig = jnp.int32(w_cnt)
    r2 = np.float32(radius * radius)

    def first_point(words):
        nz = words != 0
        sel = jnp.min(jnp.where(nz, lane_w, wbig), axis=1, keepdims=True)
        onehot = lane_w == sel
        wsel = jnp.sum(jnp.where(onehot, words, 0), axis=1, keepdims=True)
        bit = jnp.bitwise_and(wsel, -wsel)
        fb = bit.astype(jnp.float32)
        e = jnp.bitwise_and(
            jax.lax.shift_right_logical(
                jax.lax.bitcast_convert_type(fb, jnp.int32), 23), 255) - 127
        pidx = sel.astype(jnp.float32) * 32.0 + e.astype(jnp.float32)
        return pidx, bit, onehot, sel < wbig

    for c0 in range(0, m, mc):
        cent = jax.lax.slice(cent_rows, (c0, 0), (c0 + mc, 3))
        cc = jnp.sum(cent * cent, axis=1, keepdims=True)
        ab = jax.lax.dot_general(
            cent, x, (((1,), (0,)), ((), ())),
            preferred_element_type=jnp.float32)
        d2 = jnp.maximum(cc + xx - 2.0 * ab, 0.0)
        mask = (d2 <= r2).astype(jnp.float32)
        lo = jnp.dot(mask, plo_v,
                     preferred_element_type=jnp.float32).astype(jnp.int32)
        hi = jnp.dot(mask, phi_v,
                     preferred_element_type=jnp.float32).astype(jnp.int32)
        words = jnp.bitwise_or(lo, jax.lax.shift_left(hi, 16))

        v0, _, _, _ = first_point(words)
        slots0 = jnp.zeros((mc, k), jnp.float32) + v0

        def cond(carry):
            kk, go, _, _ = carry
            return (kk < k) & go

        def kstep(carry):
            kk, _, words, slots = carry
            pidx, bit, onehot, valid = first_point(words)
            slots = jnp.where((lane_k == kk) & valid, pidx, slots)
            words = jnp.bitwise_xor(
                words, jnp.where(onehot, bit, jnp.int32(0)))
            go = jnp.max(jnp.where(words != 0, 1, 0)) > 0
            return kk + 1, go, words, slots

        _, _, _, slots = jax.lax.while_loop(
            cond, kstep, (jnp.int32(0), jnp.bool_(True), words, slots0))
        gidx_ref[pl.ds(c0, mc), :] = slots


def _bq_kernel(n, m, radius, k, mc, xyz_t, cent_rows_in, plo, phi, gidx_ref):
    cent_rows = cent_rows_in[...]
    _ball_query_body(xyz_t, cent_rows, plo, phi, n, m, radius, k, mc,
                     gidx_ref)


def _mlp_kernel(m, k, g_tile, grouped, cent_rows_in, w1, w2, w3, feat_ref):
    w1v, w2v, w3v = w1[...], w2[...], w3[...]
    w1_3 = w1v[:3, :]
    rows = g_tile * k

    def tile(t, _):
        t0 = pl.multiple_of(t * g_tile, g_tile)
        r0 = pl.multiple_of(t * rows, rows)
        g = grouped[pl.ds(r0, rows), :]
        cent = cent_rows_in[pl.ds(t0, g_tile), :]
        cw = jnp.dot(cent, w1_3, preferred_element_type=jnp.float32)
        h = jnp.dot(g, w1v, preferred_element_type=jnp.float32)
        h = h.reshape(g_tile, k, -1) - cw[:, None, :]
        h = jax.nn.relu(h * _BN).reshape(rows, -1)
        h = jax.nn.relu(jnp.dot(h, w2v, preferred_element_type=jnp.float32) * _BN)
        h = jax.nn.relu(jnp.dot(h, w3v, preferred_element_type=jnp.float32) * _BN)
        h = jnp.max(h.reshape(g_tile, k, -1), axis=1)
        feat_ref[pl.ds(t0, g_tile), :] = h
        return 0

    jax.lax.fori_loop(0, m // g_tile, tile, 0)


def _fp_kernel(s, d, nw, head, sxyz_rows, dxyz_t, dfeat, sfeat, *refs):
    ws_refs, out_ref = refs[:-1], refs[-1]
    sx = sxyz_rows[...]
    dx = dxyz_t[...]
    cc_s = jnp.sum(sx * sx, axis=1, keepdims=True)
    cc_d = jnp.sum(dx * dx, axis=0, keepdims=True)
    ab = jax.lax.dot_general(sx, dx, (((1,), (0,)), ((), ())),
                             preferred_element_type=jnp.float32)
    d2 = jnp.maximum(cc_s + cc_d - 2.0 * ab, 0.0)
    lane_iota = _iota_f32((1, d), 1)
    big = np.float32(d)
    vs, idxs = [], []
    for _ in range(3):
        v, idx = _first_min_idx(d2, lane_iota, big)
        vs.append(v)
        idxs.append(idx)
        d2 = jnp.where(lane_iota == idx, np.float32(1e9), d2)
    ws = [1.0 / (v + np.float32(1e-8)) for v in vs]
    wsum = ws[0] + ws[1] + ws[2]
    df = dfeat[...]
    interp = None
    for v_idx, w in zip(idxs, ws):
        oh = (lane_iota == v_idx).astype(jnp.float32)
        gathered = jnp.dot(oh, df, preferred_element_type=jnp.float32)
        term = gathered * (w / wsum)
        interp = term if interp is None else interp + term
    f = jnp.concatenate([interp, sfeat[...]], axis=1)
    for i in range(nw):
        wv = ws_refs[i][...]
        f = jax.nn.relu(jnp.dot(f, wv, preferred_element_type=jnp.float32) * _BN)
    if head:
        seg_w, logit_w, logit_b = ws_refs[nw][...], ws_refs[nw + 1][...], ws_refs[nw + 2][...]
        f = jax.nn.relu(jnp.dot(f, seg_w, preferred_element_type=jnp.float32) * _BN)
        f = jnp.dot(f, logit_w, preferred_element_type=jnp.float32) + logit_b
    out_ref[...] = f


def _run_sa(level, xyz_t, xsrc_rows, cent_rows, w1, w2, w3):
    m, radius, k, n = *_SA_CFG[level], xyz_t.shape[-1]
    b = xyz_t.shape[0]
    g_tile = 64 if level == 0 else 16
    mc = min(m, 256)
    c3 = w3.shape[-1]

    w_cnt = n // 32
    pts = np.arange(n)
    plo_np = np.zeros((n, w_cnt), np.float32)
    phi_np = np.zeros((n, w_cnt), np.float32)
    sub = pts % 32
    lo_sel = sub < 16
    plo_np[pts[lo_sel], pts[lo_sel] // 32] = (2.0 ** sub[lo_sel])
    phi_np[pts[~lo_sel], pts[~lo_sel] // 32] = (2.0 ** (sub[~lo_sel] - 16))
    bq = pl.pallas_call(
        functools.partial(_bq_kernel, n, m, radius, k, mc),
        out_shape=jax.ShapeDtypeStruct((m, k), jnp.float32),
    )
    gidx = jax.vmap(bq, in_axes=(0, 0, None, None))(
        xyz_t, cent_rows, jnp.asarray(plo_np), jnp.asarray(phi_np))

    cs = xsrc_rows.shape[-1]
    cp = ((cs + 15) // 16) * 16
    table = jnp.pad(xsrc_rows, ((0, 0), (0, 0), (0, cp - cs)))
    table = table.reshape(b * n, cp)
    idx = (gidx.reshape(b, m * k).astype(jnp.int32)
           + (jnp.arange(b, dtype=jnp.int32) * n)[:, None]).reshape(-1)
    grouped = _sc_gather(table, idx).reshape(b, m * k, cp)

    w1p = jnp.pad(w1, ((0, cp - cs), (0, 0)))
    mlp = pl.pallas_call(
        functools.partial(_mlp_kernel, m, k, g_tile),
        out_shape=jax.ShapeDtypeStruct((m, c3), jnp.float32),
    )
    return jax.vmap(mlp, in_axes=(0, 0, None, None, None))(
        grouped, cent_rows, w1p, w2, w3)


def _run_fp(sxyz_rows, dxyz_t, dfeat, sfeat, ws, head_ws=None):
    s, d = sxyz_rows.shape[-2], dxyz_t.shape[-1]
    nw = len(ws)
    head = head_ws is not None
    all_ws = list(ws) + (list(head_ws) if head else [])
    cout = 13 if head else ws[-1].shape[-1]
    fn = functools.partial(_fp_kernel, s, d, nw, head)
    call = pl.pallas_call(
        fn,
        out_shape=jax.ShapeDtypeStruct((s, cout), jnp.float32),
    )
    in_axes = (0, 0, 0, 0) + (None,) * len(all_ws)
    return jax.vmap(call, in_axes=in_axes)(
        sxyz_rows, dxyz_t, dfeat, sfeat, *all_ws)


def kernel(points, params):
    xyz_rows = jnp.transpose(points[:, 0:3, :], (0, 2, 1))
    xyz_t = points[:, 0:3, :]
    feat_rows = jnp.transpose(points[:, 3:, :], (0, 2, 1))

    inter_xyz_rows = [xyz_rows]
    inter_xyz_t = [xyz_t]
    inter_feat = [jnp.transpose(points, (0, 2, 1))]

    cur_xyz_t, cur_xyz_rows, cur_feat = xyz_t, xyz_rows, feat_rows
    for level in range(4):
        xsrc_rows = jnp.concatenate([cur_xyz_rows, cur_feat], axis=-1)
        w1, w2, w3 = params['sa'][level]
        cent_t, cent_rows = _run_fps(level, cur_xyz_t)
        feat = _run_sa(level, cur_xyz_t, xsrc_rows, cent_rows, w1, w2, w3)
        cur_xyz_t, cur_xyz_rows, cur_feat = cent_t, cent_rows, feat
        inter_xyz_rows.append(cent_rows)
        inter_xyz_t.append(cent_t)
        inter_feat.append(feat)

    dfeat = jnp.concatenate([cur_xyz_rows, cur_feat], axis=-1)
    dxyz_t = cur_xyz_t
    for i in range(4):
        sxyz_rows = inter_xyz_rows[-1 - i]
        sfeat = inter_feat[-1 - i]
        head_ws = None
        if i == 3:
            head_ws = [params['seg'][0], params['logit_w'],
                       params['logit_b'].reshape(1, 13)]
        dfeat = _run_fp(sxyz_rows, dxyz_t, dfeat, sfeat, params['fp'][i],
                        head_ws)
        dxyz_t = inter_xyz_t[-1 - i]
    return jnp.transpose(dfeat, (0, 2, 1))
